# baseline (speedup 1.0000x reference)
import functools

import jax
import jax.numpy as jnp
from jax import lax
from jax.experimental import pallas as pl
from jax.experimental.pallas import tpu as pltpu
from jax.experimental.pallas import tpu_sc as plsc

_H_OUT = 256
_BLK = 1000
_CHUNK = 64
_NBUF = 4


def _tc_body(cf_ref, cv_ref, w2_ref, ws_ref, wv_ref, b_ref, out_ref):
    y = jnp.dot(cv_ref[...], w2_ref[...], preferred_element_type=jnp.float32)
    va = y[:, 0:64]
    vb = y[:, 64:128]
    vc = y[:, 128:192]
    vn = jnp.sqrt(va * va + vb * vb + vc * vc + 1e-8)
    acc = jnp.dot(cf_ref[...], ws_ref[...], preferred_element_type=jnp.float32)
    acc = acc + jnp.dot(vn, wv_ref[...], preferred_element_type=jnp.float32)
    acc = acc + b_ref[...]
    out_ref[...] = jnp.maximum(acc, 0.0)


def _tc_embed(cf, cv9, w2, ws, wv, b, n):
    s_in = cf.shape[1]
    fv = cv9.shape[1]
    h3 = w2.shape[1]
    h_vec = wv.shape[0]
    grid = n // _BLK
    return pl.pallas_call(
        _tc_body,
        grid=(grid,),
        in_specs=[
            pl.BlockSpec((_BLK, s_in), lambda i: (i, 0)),
            pl.BlockSpec((_BLK, fv), lambda i: (i, 0)),
            pl.BlockSpec((fv, h3), lambda i: (0, 0)),
            pl.BlockSpec((s_in, _H_OUT), lambda i: (0, 0)),
            pl.BlockSpec((h_vec, _H_OUT), lambda i: (0, 0)),
            pl.BlockSpec((1, _H_OUT), lambda i: (0, 0)),
        ],
        out_specs=pl.BlockSpec((_BLK, _H_OUT), lambda i: (i, 0)),
        out_shape=jax.ShapeDtypeStruct((n, _H_OUT), jnp.float32),
    )(cf, cv9, w2, ws, wv, b)


def _sc_gather(tbl, idx, p_out):
    info = plsc.get_sparse_core_info()
    nc, ns = info.num_cores, info.num_subcores
    nw = nc * ns
    pp = idx.shape[0]
    ppw = pp // nw

    mesh = plsc.VectorSubcoreMesh(core_axis_name="c", subcore_axis_name="s")

    @functools.partial(
        pl.kernel,
        mesh=mesh,
        out_type=jax.ShapeDtypeStruct((p_out, _H_OUT), jnp.float32),
        scratch_types=[
            pltpu.VMEM((ppw,), jnp.int32),
            pltpu.VMEM((_NBUF, _CHUNK, _H_OUT), jnp.float32),
            pltpu.SemaphoreType.DMA,
            pltpu.SemaphoreType.DMA,
            pltpu.SemaphoreType.DMA,
            pltpu.SemaphoreType.DMA,
            pltpu.SemaphoreType.DMA,
        ],
    )
    def gather_kernel(tbl_hbm, idx_hbm, out_hbm, idx_v, rows_v,
                      g0, g1, g2, g3, osem):
        wid = lax.axis_index("s") * nc + lax.axis_index("c")
        base = wid * ppw
        pltpu.sync_copy(idx_hbm.at[pl.ds(base, ppw)], idx_v)
        nch = ppw // _CHUNK
        gsems = (g0, g1, g2, g3)

        def live(j):
            return base + j * _CHUNK < p_out

        def start_gather(j):
            b = j % _NBUF

            @pl.when(live(j))
            def _():
                pltpu.async_copy(
                    tbl_hbm.at[idx_v.at[pl.ds(j * _CHUNK, _CHUNK)]],
                    rows_v.at[b],
                    gsems[b],
                )

        def wait_gather(j):
            b = j % _NBUF

            @pl.when(live(j))
            def _():
                pltpu.make_async_copy(
                    tbl_hbm.at[idx_v.at[pl.ds(j * _CHUNK, _CHUNK)]],
                    rows_v.at[b],
                    gsems[b],
                ).wait()

        def drain(j):
            b = j % _NBUF

            @pl.when(live(j))
            def _():
                pltpu.async_copy(
                    rows_v.at[b],
                    out_hbm.at[pl.ds(base + j * _CHUNK, _CHUNK)],
                    osem,
                ).wait()

        for j in range(min(_NBUF, nch)):
            start_gather(j)
        for j in range(nch):
            wait_gather(j)
            drain(j)
            if j + _NBUF < nch:
                start_gather(j + _NBUF)

    return gather_kernel(tbl, idx)


def kernel(compose_feature, compose_vec, idx_protein, Wh, Ws_w, Ws_b):
    n, s_in = compose_feature.shape
    p = idx_protein.shape[0]
    v_in = compose_vec.shape[1]
    h_vec = Wh.shape[1]

    cv9 = compose_vec.reshape(n, 3 * v_in)

    w2 = (jnp.eye(3, dtype=jnp.float32)[None, :, :, None]
          * Wh[:, None, None, :]).reshape(3 * v_in, 3 * h_vec)
    ws = Ws_w[:s_in]
    wv = Ws_w[s_in:]
    b = Ws_b.reshape(1, _H_OUT)

    h_full = _tc_embed(compose_feature, cv9, w2, ws, wv, b, n)

    align = _CHUNK * 32
    pp = -(-p // align) * align
    idx_pad = jnp.concatenate(
        [idx_protein, jnp.arange(pp - p, dtype=jnp.int32)])

    return _sc_gather(h_full, idx_pad, p)


# speedup vs baseline: 1.2898x; 1.2898x over previous
import functools

import jax
import jax.numpy as jnp
from jax import lax
from jax.experimental import pallas as pl
from jax.experimental.pallas import tpu as pltpu
from jax.experimental.pallas import tpu_sc as plsc

_H_OUT = 256
_BLK = 1024
_CHUNK = 64
_NBUF = 4


def _kt_dot(a_t, w):
    return lax.dot_general(a_t, w, (((0,), (0,)), ((), ())),
                           preferred_element_type=jnp.float32)


def _tc_body(cft_ref, cvt_ref, w2_ref, ws_ref, wv_ref, b_ref, out_ref):
    y = _kt_dot(cvt_ref[...], w2_ref[...])
    va = y[:, 0:64]
    vb = y[:, 64:128]
    vc = y[:, 128:192]
    vn = jnp.sqrt(va * va + vb * vb + vc * vc + 1e-8)
    acc = _kt_dot(cft_ref[...], ws_ref[...])
    acc = acc + jnp.dot(vn, wv_ref[...], preferred_element_type=jnp.float32)
    acc = acc + b_ref[...]
    out_ref[...] = jnp.maximum(acc, 0.0)


def _tc_embed(cft, cvt9, w2, ws, wv, b, n):
    s_in = cft.shape[0]
    fv = cvt9.shape[0]
    h3 = w2.shape[1]
    h_vec = wv.shape[0]
    grid = n // _BLK
    return pl.pallas_call(
        _tc_body,
        grid=(grid,),
        in_specs=[
            pl.BlockSpec((s_in, _BLK), lambda i: (0, i)),
            pl.BlockSpec((fv, _BLK), lambda i: (0, i)),
            pl.BlockSpec((fv, h3), lambda i: (0, 0)),
            pl.BlockSpec((s_in, _H_OUT), lambda i: (0, 0)),
            pl.BlockSpec((h_vec, _H_OUT), lambda i: (0, 0)),
            pl.BlockSpec((1, _H_OUT), lambda i: (0, 0)),
        ],
        out_specs=pl.BlockSpec((_BLK, _H_OUT), lambda i: (i, 0)),
        out_shape=jax.ShapeDtypeStruct((n, _H_OUT), jnp.float32),
    )(cft, cvt9, w2, ws, wv, b)


def _sc_gather(tbl, idx, p_out):
    info = plsc.get_sparse_core_info()
    nc, ns = info.num_cores, info.num_subcores
    nw = nc * ns
    pp = idx.shape[0]
    ppw = pp // nw

    mesh = plsc.VectorSubcoreMesh(core_axis_name="c", subcore_axis_name="s")

    @functools.partial(
        pl.kernel,
        mesh=mesh,
        out_type=jax.ShapeDtypeStruct((p_out, _H_OUT), jnp.float32),
        scratch_types=[
            pltpu.VMEM((ppw,), jnp.int32),
            pltpu.VMEM((_NBUF, _CHUNK, _H_OUT), jnp.float32),
            pltpu.SemaphoreType.DMA,
            pltpu.SemaphoreType.DMA,
            pltpu.SemaphoreType.DMA,
            pltpu.SemaphoreType.DMA,
            pltpu.SemaphoreType.DMA,
        ],
    )
    def gather_kernel(tbl_hbm, idx_hbm, out_hbm, idx_v, rows_v,
                      g0, g1, g2, g3, osem):
        wid = lax.axis_index("s") * nc + lax.axis_index("c")
        base = wid * ppw
        pltpu.sync_copy(idx_hbm.at[pl.ds(base, ppw)], idx_v)
        nch = ppw // _CHUNK
        gsems = (g0, g1, g2, g3)

        def live(j):
            return base + j * _CHUNK < p_out

        def start_gather(j):
            b = j % _NBUF

            @pl.when(live(j))
            def _():
                pltpu.async_copy(
                    tbl_hbm.at[idx_v.at[pl.ds(j * _CHUNK, _CHUNK)]],
                    rows_v.at[b],
                    gsems[b],
                )

        def wait_gather(j):
            b = j % _NBUF

            @pl.when(live(j))
            def _():
                pltpu.make_async_copy(
                    tbl_hbm.at[idx_v.at[pl.ds(j * _CHUNK, _CHUNK)]],
                    rows_v.at[b],
                    gsems[b],
                ).wait()

        def drain(j):
            b = j % _NBUF

            @pl.when(live(j))
            def _():
                pltpu.async_copy(
                    rows_v.at[b],
                    out_hbm.at[pl.ds(base + j * _CHUNK, _CHUNK)],
                    osem,
                ).wait()

        for j in range(min(_NBUF, nch)):
            start_gather(j)
        for j in range(nch):
            wait_gather(j)
            drain(j)
            if j + _NBUF < nch:
                start_gather(j + _NBUF)

    return gather_kernel(tbl, idx)


def kernel(compose_feature, compose_vec, idx_protein, Wh, Ws_w, Ws_b):
    n, s_in = compose_feature.shape
    p = idx_protein.shape[0]
    v_in = compose_vec.shape[1]
    h_vec = Wh.shape[1]

    npad = -(-n // _BLK) * _BLK
    cft = jnp.pad(compose_feature.T, ((0, 0), (0, npad - n)))
    cvt9 = jnp.pad(
        compose_vec.transpose(1, 2, 0).reshape(3 * v_in, n),
        ((0, 0), (0, npad - n)))

    w2 = (jnp.eye(3, dtype=jnp.float32)[None, :, :, None]
          * Wh[:, None, None, :]).reshape(3 * v_in, 3 * h_vec)
    ws = Ws_w[:s_in]
    wv = Ws_w[s_in:]
    b = Ws_b.reshape(1, _H_OUT)

    h_full = _tc_embed(cft, cvt9, w2, ws, wv, b, npad)

    align = _CHUNK * 32
    pp = -(-p // align) * align
    idx_pad = jnp.concatenate(
        [idx_protein, jnp.arange(pp - p, dtype=jnp.int32)])

    return _sc_gather(h_full, idx_pad, p)


# speedup vs baseline: 1.2989x; 1.0071x over previous
import functools

import jax
import jax.numpy as jnp
from jax import lax
from jax.experimental import pallas as pl
from jax.experimental.pallas import tpu as pltpu
from jax.experimental.pallas import tpu_sc as plsc

_H_OUT = 256
_BLK = 1024
_CHUNK = 64
_NBUF = 6


def _kt_dot(a_t, w):
    return lax.dot_general(a_t, w, (((0,), (0,)), ((), ())),
                           preferred_element_type=jnp.float32)


def _tc_body(cft_ref, cvt_ref, w2_ref, ws_ref, wv_ref, b_ref, out_ref):
    y = _kt_dot(cvt_ref[...], w2_ref[...])
    va = y[:, 0:64]
    vb = y[:, 64:128]
    vc = y[:, 128:192]
    vn = jnp.sqrt(va * va + vb * vb + vc * vc + 1e-8)
    acc = _kt_dot(cft_ref[...], ws_ref[...])
    acc = acc + jnp.dot(vn, wv_ref[...], preferred_element_type=jnp.float32)
    acc = acc + b_ref[...]
    out_ref[...] = jnp.maximum(acc, 0.0)


def _tc_embed(cft, cvt9, w2, ws, wv, b, n):
    s_in = cft.shape[0]
    fv = cvt9.shape[0]
    h3 = w2.shape[1]
    h_vec = wv.shape[0]
    grid = n // _BLK
    return pl.pallas_call(
        _tc_body,
        grid=(grid,),
        in_specs=[
            pl.BlockSpec((s_in, _BLK), lambda i: (0, i)),
            pl.BlockSpec((fv, _BLK), lambda i: (0, i)),
            pl.BlockSpec((fv, h3), lambda i: (0, 0)),
            pl.BlockSpec((s_in, _H_OUT), lambda i: (0, 0)),
            pl.BlockSpec((h_vec, _H_OUT), lambda i: (0, 0)),
            pl.BlockSpec((1, _H_OUT), lambda i: (0, 0)),
        ],
        out_specs=pl.BlockSpec((_BLK, _H_OUT), lambda i: (i, 0)),
        out_shape=jax.ShapeDtypeStruct((n, _H_OUT), jnp.float32),
        compiler_params=pltpu.CompilerParams(
            fuse_transposed_lhs_in_matmul=True),
    )(cft, cvt9, w2, ws, wv, b)


def _sc_gather(tbl, idx, p_out):
    info = plsc.get_sparse_core_info()
    nc, ns = info.num_cores, info.num_subcores
    nw = nc * ns
    pp = idx.shape[0]
    ppw = pp // nw

    mesh = plsc.VectorSubcoreMesh(core_axis_name="c", subcore_axis_name="s")

    @functools.partial(
        pl.kernel,
        mesh=mesh,
        out_type=jax.ShapeDtypeStruct((p_out, _H_OUT), jnp.float32),
        scratch_types=[
            pltpu.VMEM((ppw,), jnp.int32),
            pltpu.VMEM((_NBUF, _CHUNK, _H_OUT), jnp.float32),
        ] + [pltpu.SemaphoreType.DMA] * (_NBUF + 1),
    )
    def gather_kernel(tbl_hbm, idx_hbm, out_hbm, idx_v, rows_v, *sems):
        wid = lax.axis_index("s") * nc + lax.axis_index("c")
        base = wid * ppw
        pltpu.sync_copy(idx_hbm.at[pl.ds(base, ppw)], idx_v)
        nch = ppw // _CHUNK
        gsems = sems[:_NBUF]
        osem = sems[_NBUF]

        def live(j):
            return base + j * _CHUNK < p_out

        def start_gather(j):
            b = j % _NBUF

            @pl.when(live(j))
            def _():
                pltpu.async_copy(
                    tbl_hbm.at[idx_v.at[pl.ds(j * _CHUNK, _CHUNK)]],
                    rows_v.at[b],
                    gsems[b],
                )

        def wait_gather(j):
            b = j % _NBUF

            @pl.when(live(j))
            def _():
                pltpu.make_async_copy(
                    tbl_hbm.at[idx_v.at[pl.ds(j * _CHUNK, _CHUNK)]],
                    rows_v.at[b],
                    gsems[b],
                ).wait()

        def drain(j):
            b = j % _NBUF

            @pl.when(live(j))
            def _():
                pltpu.async_copy(
                    rows_v.at[b],
                    out_hbm.at[pl.ds(base + j * _CHUNK, _CHUNK)],
                    osem,
                ).wait()

        for j in range(min(_NBUF, nch)):
            start_gather(j)
        for j in range(nch):
            wait_gather(j)
            drain(j)
            if j + _NBUF < nch:
                start_gather(j + _NBUF)

    return gather_kernel(tbl, idx)


def kernel(compose_feature, compose_vec, idx_protein, Wh, Ws_w, Ws_b):
    n, s_in = compose_feature.shape
    p = idx_protein.shape[0]
    v_in = compose_vec.shape[1]
    h_vec = Wh.shape[1]

    npad = -(-n // _BLK) * _BLK
    cft = jnp.pad(compose_feature.T, ((0, 0), (0, npad - n)))
    cvt9 = jnp.pad(
        compose_vec.transpose(1, 2, 0).reshape(3 * v_in, n),
        ((0, 0), (0, npad - n)))

    w2 = (jnp.eye(3, dtype=jnp.float32)[None, :, :, None]
          * Wh[:, None, None, :]).reshape(3 * v_in, 3 * h_vec)
    ws = Ws_w[:s_in]
    wv = Ws_w[s_in:]
    b = Ws_b.reshape(1, _H_OUT)

    h_full = _tc_embed(cft, cvt9, w2, ws, wv, b, npad)

    align = _CHUNK * 32
    pp = -(-p // align) * align
    idx_pad = jnp.concatenate(
        [idx_protein, jnp.arange(pp - p, dtype=jnp.int32)])

    return _sc_gather(h_full, idx_pad, p)


# speedup vs baseline: 1.4285x; 1.0997x over previous
import functools

import jax
import jax.numpy as jnp
from jax import lax
from jax.experimental import pallas as pl
from jax.experimental.pallas import tpu as pltpu
from jax.experimental.pallas import tpu_sc as plsc

_H_OUT = 256
_BLK = 2048
_CHUNK = 64
_NBUF = 6


def _kt_dot(a_t, w):
    return lax.dot_general(a_t, w, (((0,), (0,)), ((), ())),
                           preferred_element_type=jnp.float32)


def _tc_body(cft_ref, cvt_ref, w2_ref, ws_ref, wv_ref, b_ref, out_ref):
    y = _kt_dot(cvt_ref[...], w2_ref[...])
    va = y[:, 0:64]
    vb = y[:, 64:128]
    vc = y[:, 128:192]
    vn = jnp.sqrt(va * va + vb * vb + vc * vc + 1e-8)
    acc = _kt_dot(cft_ref[...], ws_ref[...])
    acc = acc + jnp.dot(vn, wv_ref[...], preferred_element_type=jnp.float32)
    acc = acc + b_ref[...]
    out_ref[...] = jnp.maximum(acc, 0.0)


def _tc_embed(cft, cvt9, w2, ws, wv, b, n):
    s_in = cft.shape[0]
    fv = cvt9.shape[0]
    h3 = w2.shape[1]
    h_vec = wv.shape[0]
    grid = n // _BLK
    return pl.pallas_call(
        _tc_body,
        grid=(grid,),
        in_specs=[
            pl.BlockSpec((s_in, _BLK), lambda i: (0, i)),
            pl.BlockSpec((fv, _BLK), lambda i: (0, i)),
            pl.BlockSpec((fv, h3), lambda i: (0, 0)),
            pl.BlockSpec((s_in, _H_OUT), lambda i: (0, 0)),
            pl.BlockSpec((h_vec, _H_OUT), lambda i: (0, 0)),
            pl.BlockSpec((1, _H_OUT), lambda i: (0, 0)),
        ],
        out_specs=pl.BlockSpec((_BLK, _H_OUT), lambda i: (i, 0)),
        out_shape=jax.ShapeDtypeStruct((n, _H_OUT), jnp.float32),
        compiler_params=pltpu.CompilerParams(
            fuse_transposed_lhs_in_matmul=True),
    )(cft, cvt9, w2, ws, wv, b)


def _sc_gather(tbl, idx, p_out):
    info = plsc.get_sparse_core_info()
    nc, ns = info.num_cores, info.num_subcores
    nw = nc * ns
    pp = idx.shape[0]
    ppw = pp // nw

    mesh = plsc.VectorSubcoreMesh(core_axis_name="c", subcore_axis_name="s")

    @functools.partial(
        pl.kernel,
        mesh=mesh,
        out_type=jax.ShapeDtypeStruct((p_out, _H_OUT), jnp.float32),
        scratch_types=[
            pltpu.VMEM((ppw,), jnp.int32),
            pltpu.VMEM((_NBUF, _CHUNK, _H_OUT), jnp.float32),
        ] + [pltpu.SemaphoreType.DMA] * (_NBUF + 1),
    )
    def gather_kernel(tbl_hbm, idx_hbm, out_hbm, idx_v, rows_v, *sems):
        wid = lax.axis_index("s") * nc + lax.axis_index("c")
        base = wid * ppw
        pltpu.sync_copy(idx_hbm.at[pl.ds(base, ppw)], idx_v)
        nch = ppw // _CHUNK
        gsems = sems[:_NBUF]
        osem = sems[_NBUF]

        def live(j):
            return base + j * _CHUNK < p_out

        def start_gather(j):
            b = j % _NBUF

            @pl.when(live(j))
            def _():
                pltpu.async_copy(
                    tbl_hbm.at[idx_v.at[pl.ds(j * _CHUNK, _CHUNK)]],
                    rows_v.at[b],
                    gsems[b],
                )

        def wait_gather(j):
            b = j % _NBUF

            @pl.when(live(j))
            def _():
                pltpu.make_async_copy(
                    tbl_hbm.at[idx_v.at[pl.ds(j * _CHUNK, _CHUNK)]],
                    rows_v.at[b],
                    gsems[b],
                ).wait()

        def drain(j):
            b = j % _NBUF

            @pl.when(live(j))
            def _():
                pltpu.async_copy(
                    rows_v.at[b],
                    out_hbm.at[pl.ds(base + j * _CHUNK, _CHUNK)],
                    osem,
                ).wait()

        for j in range(min(_NBUF, nch)):
            start_gather(j)
        for j in range(nch):
            wait_gather(j)
            drain(j)
            if j + _NBUF < nch:
                start_gather(j + _NBUF)

    return gather_kernel(tbl, idx)


def kernel(compose_feature, compose_vec, idx_protein, Wh, Ws_w, Ws_b):
    n, s_in = compose_feature.shape
    p = idx_protein.shape[0]
    v_in = compose_vec.shape[1]
    h_vec = Wh.shape[1]

    npad = -(-n // _BLK) * _BLK
    cft = jnp.pad(compose_feature.T, ((0, 0), (0, npad - n)))
    cvt9 = jnp.pad(
        compose_vec.transpose(1, 2, 0).reshape(3 * v_in, n),
        ((0, 0), (0, npad - n)))

    w2 = (jnp.eye(3, dtype=jnp.float32)[None, :, :, None]
          * Wh[:, None, None, :]).reshape(3 * v_in, 3 * h_vec)
    ws = Ws_w[:s_in]
    wv = Ws_w[s_in:]
    b = Ws_b.reshape(1, _H_OUT)

    h_full = _tc_embed(cft, cvt9, w2, ws, wv, b, npad)

    align = _CHUNK * 32
    pp = -(-p // align) * align
    idx_pad = jnp.concatenate(
        [idx_protein, jnp.arange(pp - p, dtype=jnp.int32)])

    return _sc_gather(h_full, idx_pad, p)


# speedup vs baseline: 1.4563x; 1.0195x over previous
import functools

import jax
import jax.numpy as jnp
from jax import lax
from jax.experimental import pallas as pl
from jax.experimental.pallas import tpu as pltpu
from jax.experimental.pallas import tpu_sc as plsc

_H_OUT = 256
_BLK = 4096
_CHUNK = 64
_NBUF = 6


def _kt_dot(a_t, w):
    return lax.dot_general(a_t, w, (((0,), (0,)), ((), ())),
                           preferred_element_type=jnp.float32)


def _tc_body(cft_ref, cvt_ref, w2_ref, ws_ref, wv_ref, b_ref, out_ref):
    y = _kt_dot(cvt_ref[...], w2_ref[...])
    va = y[:, 0:64]
    vb = y[:, 64:128]
    vc = y[:, 128:192]
    vn = jnp.sqrt(va * va + vb * vb + vc * vc + 1e-8)
    acc = _kt_dot(cft_ref[...], ws_ref[...])
    acc = acc + jnp.dot(vn, wv_ref[...], preferred_element_type=jnp.float32)
    acc = acc + b_ref[...]
    out_ref[...] = jnp.maximum(acc, 0.0)


def _tc_embed(cft, cvt9, w2, ws, wv, b, n):
    s_in = cft.shape[0]
    fv = cvt9.shape[0]
    h3 = w2.shape[1]
    h_vec = wv.shape[0]
    grid = n // _BLK
    return pl.pallas_call(
        _tc_body,
        grid=(grid,),
        in_specs=[
            pl.BlockSpec((s_in, _BLK), lambda i: (0, i)),
            pl.BlockSpec((fv, _BLK), lambda i: (0, i)),
            pl.BlockSpec((fv, h3), lambda i: (0, 0)),
            pl.BlockSpec((s_in, _H_OUT), lambda i: (0, 0)),
            pl.BlockSpec((h_vec, _H_OUT), lambda i: (0, 0)),
            pl.BlockSpec((1, _H_OUT), lambda i: (0, 0)),
        ],
        out_specs=pl.BlockSpec((_BLK, _H_OUT), lambda i: (i, 0)),
        out_shape=jax.ShapeDtypeStruct((n, _H_OUT), jnp.float32),
        compiler_params=pltpu.CompilerParams(
            fuse_transposed_lhs_in_matmul=True),
    )(cft, cvt9, w2, ws, wv, b)


def _sc_gather(tbl, idx, p_out):
    info = plsc.get_sparse_core_info()
    nc, ns = info.num_cores, info.num_subcores
    nw = nc * ns
    pp = idx.shape[0]
    ppw = pp // nw

    mesh = plsc.VectorSubcoreMesh(core_axis_name="c", subcore_axis_name="s")

    @functools.partial(
        pl.kernel,
        mesh=mesh,
        out_type=jax.ShapeDtypeStruct((p_out, _H_OUT), jnp.float32),
        scratch_types=[
            pltpu.VMEM((ppw,), jnp.int32),
            pltpu.VMEM((_NBUF, _CHUNK, _H_OUT), jnp.float32),
        ] + [pltpu.SemaphoreType.DMA] * (_NBUF + 1),
    )
    def gather_kernel(tbl_hbm, idx_hbm, out_hbm, idx_v, rows_v, *sems):
        wid = lax.axis_index("s") * nc + lax.axis_index("c")
        base = wid * ppw
        pltpu.sync_copy(idx_hbm.at[pl.ds(base, ppw)], idx_v)
        nch = ppw // _CHUNK
        gsems = sems[:_NBUF]
        osem = sems[_NBUF]

        def live(j):
            return base + j * _CHUNK < p_out

        def start_gather(j):
            b = j % _NBUF

            @pl.when(live(j))
            def _():
                pltpu.async_copy(
                    tbl_hbm.at[idx_v.at[pl.ds(j * _CHUNK, _CHUNK)]],
                    rows_v.at[b],
                    gsems[b],
                )

        def wait_gather(j):
            b = j % _NBUF

            @pl.when(live(j))
            def _():
                pltpu.make_async_copy(
                    tbl_hbm.at[idx_v.at[pl.ds(j * _CHUNK, _CHUNK)]],
                    rows_v.at[b],
                    gsems[b],
                ).wait()

        def drain(j):
            b = j % _NBUF

            @pl.when(live(j))
            def _():
                pltpu.async_copy(
                    rows_v.at[b],
                    out_hbm.at[pl.ds(base + j * _CHUNK, _CHUNK)],
                    osem,
                ).wait()

        for j in range(min(_NBUF, nch)):
            start_gather(j)
        for j in range(nch):
            wait_gather(j)
            drain(j)
            if j + _NBUF < nch:
                start_gather(j + _NBUF)

    return gather_kernel(tbl, idx)


def kernel(compose_feature, compose_vec, idx_protein, Wh, Ws_w, Ws_b):
    n, s_in = compose_feature.shape
    p = idx_protein.shape[0]
    v_in = compose_vec.shape[1]
    h_vec = Wh.shape[1]

    npad = -(-n // _BLK) * _BLK
    cft = jnp.pad(compose_feature.T, ((0, 0), (0, npad - n)))
    cvt9 = jnp.pad(
        compose_vec.transpose(1, 2, 0).reshape(3 * v_in, n),
        ((0, 0), (0, npad - n)))

    w2 = (jnp.eye(3, dtype=jnp.float32)[None, :, :, None]
          * Wh[:, None, None, :]).reshape(3 * v_in, 3 * h_vec)
    ws = Ws_w[:s_in]
    wv = Ws_w[s_in:]
    b = Ws_b.reshape(1, _H_OUT)

    h_full = _tc_embed(cft, cvt9, w2, ws, wv, b, npad)

    align = _CHUNK * 32
    pp = -(-p // align) * align
    idx_pad = jnp.concatenate(
        [idx_protein, jnp.arange(pp - p, dtype=jnp.int32)])

    return _sc_gather(h_full, idx_pad, p)


# speedup vs baseline: 1.4622x; 1.0040x over previous
import functools

import jax
import jax.numpy as jnp
from jax import lax
from jax.experimental import pallas as pl
from jax.experimental.pallas import tpu as pltpu
from jax.experimental.pallas import tpu_sc as plsc

_H_OUT = 256
_BLK = 4096
_CHUNK = 128
_NBUF = 3


def _kt_dot(a_t, w):
    return lax.dot_general(a_t, w, (((0,), (0,)), ((), ())),
                           preferred_element_type=jnp.float32)


def _tc_body(cft_ref, cvt_ref, w2_ref, ws_ref, wv_ref, b_ref, out_ref):
    y = _kt_dot(cvt_ref[...], w2_ref[...])
    va = y[:, 0:64]
    vb = y[:, 64:128]
    vc = y[:, 128:192]
    vn = jnp.sqrt(va * va + vb * vb + vc * vc + 1e-8)
    acc = _kt_dot(cft_ref[...], ws_ref[...])
    acc = acc + jnp.dot(vn, wv_ref[...], preferred_element_type=jnp.float32)
    acc = acc + b_ref[...]
    out_ref[...] = jnp.maximum(acc, 0.0)


def _tc_embed(cft, cvt9, w2, ws, wv, b, n):
    s_in = cft.shape[0]
    fv = cvt9.shape[0]
    h3 = w2.shape[1]
    h_vec = wv.shape[0]
    grid = n // _BLK
    return pl.pallas_call(
        _tc_body,
        grid=(grid,),
        in_specs=[
            pl.BlockSpec((s_in, _BLK), lambda i: (0, i)),
            pl.BlockSpec((fv, _BLK), lambda i: (0, i)),
            pl.BlockSpec((fv, h3), lambda i: (0, 0)),
            pl.BlockSpec((s_in, _H_OUT), lambda i: (0, 0)),
            pl.BlockSpec((h_vec, _H_OUT), lambda i: (0, 0)),
            pl.BlockSpec((1, _H_OUT), lambda i: (0, 0)),
        ],
        out_specs=pl.BlockSpec((_BLK, _H_OUT), lambda i: (i, 0)),
        out_shape=jax.ShapeDtypeStruct((n, _H_OUT), jnp.float32),
        compiler_params=pltpu.CompilerParams(
            fuse_transposed_lhs_in_matmul=True),
    )(cft, cvt9, w2, ws, wv, b)


def _sc_gather(tbl, idx, p_out):
    info = plsc.get_sparse_core_info()
    nc, ns = info.num_cores, info.num_subcores
    nw = nc * ns
    pp = idx.shape[0]
    ppw = pp // nw

    mesh = plsc.VectorSubcoreMesh(core_axis_name="c", subcore_axis_name="s")

    @functools.partial(
        pl.kernel,
        mesh=mesh,
        out_type=jax.ShapeDtypeStruct((p_out, _H_OUT), jnp.float32),
        scratch_types=[
            pltpu.VMEM((ppw,), jnp.int32),
            pltpu.VMEM((_NBUF, _CHUNK, _H_OUT), jnp.float32),
        ] + [pltpu.SemaphoreType.DMA] * (_NBUF + 1),
    )
    def gather_kernel(tbl_hbm, idx_hbm, out_hbm, idx_v, rows_v, *sems):
        wid = lax.axis_index("s") * nc + lax.axis_index("c")
        base = wid * ppw
        pltpu.sync_copy(idx_hbm.at[pl.ds(base, ppw)], idx_v)
        nch = ppw // _CHUNK
        gsems = sems[:_NBUF]
        osem = sems[_NBUF]

        rem = p_out % _CHUNK

        def live(j):
            return base + j * _CHUNK < p_out

        def start_gather(j):
            b = j % _NBUF

            @pl.when(live(j))
            def _():
                pltpu.async_copy(
                    tbl_hbm.at[idx_v.at[pl.ds(j * _CHUNK, _CHUNK)]],
                    rows_v.at[b],
                    gsems[b],
                )

        def wait_gather(j):
            b = j % _NBUF

            @pl.when(live(j))
            def _():
                pltpu.make_async_copy(
                    tbl_hbm.at[idx_v.at[pl.ds(j * _CHUNK, _CHUNK)]],
                    rows_v.at[b],
                    gsems[b],
                ).wait()

        def drain(j):
            b = j % _NBUF
            start = base + j * _CHUNK

            @pl.when(start + _CHUNK <= p_out)
            def _():
                pltpu.async_copy(
                    rows_v.at[b],
                    out_hbm.at[pl.ds(start, _CHUNK)],
                    osem,
                ).wait()

            if rem:
                @pl.when(start == p_out - rem)
                def _():
                    pltpu.async_copy(
                        rows_v.at[b].at[pl.ds(0, rem)],
                        out_hbm.at[pl.ds(start, rem)],
                        osem,
                    ).wait()

        for j in range(min(_NBUF, nch)):
            start_gather(j)
        for j in range(nch):
            wait_gather(j)
            drain(j)
            if j + _NBUF < nch:
                start_gather(j + _NBUF)

    return gather_kernel(tbl, idx)


def kernel(compose_feature, compose_vec, idx_protein, Wh, Ws_w, Ws_b):
    n, s_in = compose_feature.shape
    p = idx_protein.shape[0]
    v_in = compose_vec.shape[1]
    h_vec = Wh.shape[1]

    npad = -(-n // _BLK) * _BLK
    cft = jnp.pad(compose_feature.T, ((0, 0), (0, npad - n)))
    cvt9 = jnp.pad(
        compose_vec.transpose(1, 2, 0).reshape(3 * v_in, n),
        ((0, 0), (0, npad - n)))

    w2 = (jnp.eye(3, dtype=jnp.float32)[None, :, :, None]
          * Wh[:, None, None, :]).reshape(3 * v_in, 3 * h_vec)
    ws = Ws_w[:s_in]
    wv = Ws_w[s_in:]
    b = Ws_b.reshape(1, _H_OUT)

    h_full = _tc_embed(cft, cvt9, w2, ws, wv, b, npad)

    align = _CHUNK * 32
    pp = -(-p // align) * align
    idx_pad = jnp.concatenate(
        [idx_protein, jnp.arange(pp - p, dtype=jnp.int32)])

    return _sc_gather(h_full, idx_pad, p)
